# Initial kernel scaffold; baseline (speedup 1.0000x reference)
import jax
import jax.numpy as jnp
from jax.experimental import pallas as pl


def kernel(x_, edge_index, W, att_src, att_dst, bias):
    raise NotImplementedError("write your pallas kernel here")


# speedup vs baseline: 14.7758x; 14.7758x over previous
import jax
import jax.numpy as jnp
from jax import lax
from jax.experimental import pallas as pl
from jax.experimental.pallas import tpu as pltpu
from jax.experimental.pallas import tpu_sc as plsc

N = 10000
E = 320000
D = 128
DH = 64
DP = 80
NC = 2
NS = 16
EPW = E // NS
CH = 80
NCH = EPW // CH
RPT = N // NS
ZR = 125
L = 16


def _embed_body(x_ref, w_ref, att_ref, hpad_ref, alpha_ref):
    h = jnp.dot(x_ref[...], w_ref[...], preferred_element_type=jnp.float32)
    col = lax.broadcasted_iota(jnp.int32, (N, DP - DH), 1)
    pad = jnp.where(col == 0, 1.0, 0.0).astype(jnp.float32)
    hpad_ref[0, :, :DH] = h[:, :DH]
    hpad_ref[0, :, DH:] = pad
    hpad_ref[1, :, :DH] = h[:, DH:]
    hpad_ref[1, :, DH:] = pad
    alpha_ref[...] = lax.dot_general(
        att_ref[...], h, (((1,), (1,)), ((), ())),
        preferred_element_type=jnp.float32)


def _embed(x, W, att):
    return pl.pallas_call(
        _embed_body,
        out_shape=[
            jax.ShapeDtypeStruct((NC, N, DP), jnp.float32),
            jax.ShapeDtypeStruct((2, N), jnp.float32),
        ],
    )(x, W, att)


def _edge_body(hpad_hbm, alpha_hbm, src_hbm, dst_hbm, out_hbm,
               asrc_v, adst_v, sidx_c, didx_c, w_c, rows_v, zb_v, acc_sh):
    c = lax.axis_index("c")
    s = lax.axis_index("s")

    def _zrow(r, _):
        for cc in range(DP // L):
            zb_v[r, pl.ds(cc * L, L)] = jnp.zeros((L,), jnp.float32)
        return ()
    lax.fori_loop(0, ZR, _zrow, ())
    for t in range(RPT // ZR):
        pltpu.sync_copy(zb_v, acc_sh.at[pl.ds(s * RPT + t * ZR, ZR)])

    pltpu.sync_copy(alpha_hbm.at[0], asrc_v)
    pltpu.sync_copy(alpha_hbm.at[1], adst_v)

    plsc.subcore_barrier()

    def _chunk(j, _):
        pltpu.sync_copy(src_hbm.at[s].at[j], sidx_c)
        pltpu.sync_copy(dst_hbm.at[s].at[j], didx_c)
        pltpu.sync_copy(hpad_hbm.at[c].at[sidx_c], rows_v)

        for k in range(CH // L):
            s16 = sidx_c[pl.ds(k * L, L)]
            d16 = didx_c[pl.ds(k * L, L)]
            al = (plsc.load_gather(asrc_v, [s16])
                  + plsc.load_gather(adst_v, [d16]))
            al = jnp.where(al >= 0.0, al, al * jnp.float32(0.2))
            w_c[pl.ds(k * L, L)] = jnp.exp(al)

        def _scale(r, _):
            wb = plsc.load_gather(w_c, [jnp.full((L,), r, jnp.int32)])
            for cc in range(DP // L):
                sl = pl.ds(cc * L, L)
                rows_v[r, sl] = rows_v[r, sl] * wb
            return ()
        lax.fori_loop(0, CH, _scale, ())

        pltpu.sync_copy(rows_v, acc_sh.at[didx_c], add=True)
        return ()
    lax.fori_loop(0, NCH, _chunk, ())

    plsc.subcore_barrier()

    for t in range(RPT // ZR):
        row0 = s * RPT + t * ZR
        pltpu.sync_copy(acc_sh.at[pl.ds(row0, ZR)], zb_v)
        pltpu.sync_copy(zb_v, out_hbm.at[c].at[pl.ds(row0, ZR)])


def _edge(hpad, alpha2, src_r, dst_r):
    mesh = plsc.VectorSubcoreMesh(
        core_axis_name="c", subcore_axis_name="s",
        num_cores=NC, num_subcores=NS)
    run = pl.kernel(
        _edge_body,
        out_type=jax.ShapeDtypeStruct((NC, N, DP), jnp.float32),
        mesh=mesh,
        compiler_params=pltpu.CompilerParams(
            use_tc_tiling_on_sc=False, needs_layout_passes=False),
        scratch_types=[
            pltpu.VMEM((N,), jnp.float32),
            pltpu.VMEM((N,), jnp.float32),
            pltpu.VMEM((CH,), jnp.int32),
            pltpu.VMEM((CH,), jnp.int32),
            pltpu.VMEM((CH,), jnp.float32),
            pltpu.VMEM((CH, DP), jnp.float32),
            pltpu.VMEM((ZR, DP), jnp.float32),
            pltpu.VMEM_SHARED((N, DP), jnp.float32),
        ],
    )
    return run(hpad, alpha2, src_r, dst_r)


def _combine_body(part_ref, bias_ref, out_ref):
    p = part_ref[...]
    eps = jnp.float32(1e-16)
    left = p[0, :, :DH] / (p[0, :, DH:DH + 1] + eps)
    right = p[1, :, :DH] / (p[1, :, DH:DH + 1] + eps)
    out_ref[...] = (jnp.concatenate([left, right], axis=1)
                    + bias_ref[...][None, :])


def _combine(part, bias):
    bm = 1000
    return pl.pallas_call(
        _combine_body,
        grid=(N // bm,),
        in_specs=[
            pl.BlockSpec((NC, bm, DP), lambda i: (0, i, 0)),
            pl.BlockSpec((D,), lambda i: (0,)),
        ],
        out_specs=pl.BlockSpec((bm, D), lambda i: (i, 0)),
        out_shape=jax.ShapeDtypeStruct((N, D), jnp.float32),
    )(part, bias)


def kernel(x_, edge_index, W, att_src, att_dst, bias):
    x = x_.reshape(N, D)
    src = edge_index[0].astype(jnp.int32).reshape(NS, NCH, CH)
    dst = edge_index[1].astype(jnp.int32).reshape(NS, NCH, CH)
    att = jnp.concatenate(
        [att_src.reshape(1, D), att_dst.reshape(1, D)], axis=0)

    hpad, alpha2 = _embed(x, W, att)
    part = _edge(hpad, alpha2, src, dst)
    out = _combine(part, bias)
    return out.reshape(1, N, D)


# speedup vs baseline: 24.1554x; 1.6348x over previous
import jax
import jax.numpy as jnp
from jax import lax
from jax.experimental import pallas as pl
from jax.experimental.pallas import tpu as pltpu
from jax.experimental.pallas import tpu_sc as plsc

N = 10000
E = 320000
D = 128
DH = 64
DP = 80
NC = 2
NS = 16
EPW = E // NS
CH = 400
NCH = EPW // CH
RPT = N // NS
ZR = 125
L = 16


def _embed_body(x_ref, w_ref, att_ref, hpad_ref, alpha_ref):
    h = jnp.dot(x_ref[...], w_ref[...], preferred_element_type=jnp.float32)
    col = lax.broadcasted_iota(jnp.int32, (N, DP - DH), 1)
    pad = jnp.where(col == 0, 1.0, 0.0).astype(jnp.float32)
    hpad_ref[0, :, :DH] = h[:, :DH]
    hpad_ref[0, :, DH:] = pad
    hpad_ref[1, :, :DH] = h[:, DH:]
    hpad_ref[1, :, DH:] = pad
    alpha_ref[...] = lax.dot_general(
        att_ref[...], h, (((1,), (1,)), ((), ())),
        preferred_element_type=jnp.float32)


def _embed(x, W, att):
    return pl.pallas_call(
        _embed_body,
        out_shape=[
            jax.ShapeDtypeStruct((NC, N, DP), jnp.float32),
            jax.ShapeDtypeStruct((2, N), jnp.float32),
        ],
    )(x, W, att)


def _edge_body(hpad_hbm, alpha_hbm, src_hbm, dst_hbm, out_hbm,
               asrc_v, adst_v, sidx_c, didx_c, w_c, rows_v, acc_sh):
    c = lax.axis_index("c")
    s = lax.axis_index("s")

    def _zrow(r, _):
        for cc in range(DP // L):
            rows_v[r, pl.ds(cc * L, L)] = jnp.zeros((L,), jnp.float32)
        return ()
    lax.fori_loop(0, ZR, _zrow, ())
    zb = rows_v.at[pl.ds(0, ZR)]
    for t in range(RPT // ZR):
        pltpu.sync_copy(zb, acc_sh.at[pl.ds(s * RPT + t * ZR, ZR)])

    pltpu.sync_copy(alpha_hbm.at[0], asrc_v)
    pltpu.sync_copy(alpha_hbm.at[1], adst_v)

    plsc.subcore_barrier()

    def _chunk(j, _):
        pltpu.sync_copy(src_hbm.at[s].at[j], sidx_c)
        pltpu.sync_copy(dst_hbm.at[s].at[j], didx_c)
        pltpu.sync_copy(hpad_hbm.at[c].at[sidx_c], rows_v)

        for k in range(CH // L):
            s16 = sidx_c[pl.ds(k * L, L)]
            d16 = didx_c[pl.ds(k * L, L)]
            al = (plsc.load_gather(asrc_v, [s16])
                  + plsc.load_gather(adst_v, [d16]))
            al = jnp.where(al >= 0.0, al, al * jnp.float32(0.2))
            w_c[pl.ds(k * L, L)] = jnp.exp(al)

        def _scale(r, _):
            wb = plsc.load_gather(w_c, [jnp.full((L,), r, jnp.int32)])
            for cc in range(DP // L):
                sl = pl.ds(cc * L, L)
                rows_v[r, sl] = rows_v[r, sl] * wb
            return ()
        lax.fori_loop(0, CH, _scale, ())

        pltpu.sync_copy(rows_v, acc_sh.at[didx_c], add=True)
        return ()
    lax.fori_loop(0, NCH, _chunk, ())

    plsc.subcore_barrier()

    wb = rows_v.at[pl.ds(0, ZR)]
    for t in range(RPT // ZR):
        row0 = s * RPT + t * ZR
        pltpu.sync_copy(acc_sh.at[pl.ds(row0, ZR)], wb)
        pltpu.sync_copy(wb, out_hbm.at[c].at[pl.ds(row0, ZR)])


def _edge(hpad, alpha2, src_r, dst_r):
    mesh = plsc.VectorSubcoreMesh(
        core_axis_name="c", subcore_axis_name="s",
        num_cores=NC, num_subcores=NS)
    run = pl.kernel(
        _edge_body,
        out_type=jax.ShapeDtypeStruct((NC, N, DP), jnp.float32),
        mesh=mesh,
        compiler_params=pltpu.CompilerParams(
            use_tc_tiling_on_sc=False, needs_layout_passes=False),
        scratch_types=[
            pltpu.VMEM((N,), jnp.float32),
            pltpu.VMEM((N,), jnp.float32),
            pltpu.VMEM((CH,), jnp.int32),
            pltpu.VMEM((CH,), jnp.int32),
            pltpu.VMEM((CH,), jnp.float32),
            pltpu.VMEM((CH, DP), jnp.float32),
            pltpu.VMEM_SHARED((N, DP), jnp.float32),
        ],
    )
    return run(hpad, alpha2, src_r, dst_r)


def _combine_body(part_ref, bias_ref, out_ref):
    p = part_ref[...]
    eps = jnp.float32(1e-16)
    left = p[0, :, :DH] / (p[0, :, DH:DH + 1] + eps)
    right = p[1, :, :DH] / (p[1, :, DH:DH + 1] + eps)
    out_ref[...] = (jnp.concatenate([left, right], axis=1)
                    + bias_ref[...][None, :])


def _combine(part, bias):
    bm = 1000
    return pl.pallas_call(
        _combine_body,
        grid=(N // bm,),
        in_specs=[
            pl.BlockSpec((NC, bm, DP), lambda i: (0, i, 0)),
            pl.BlockSpec((D,), lambda i: (0,)),
        ],
        out_specs=pl.BlockSpec((bm, D), lambda i: (i, 0)),
        out_shape=jax.ShapeDtypeStruct((N, D), jnp.float32),
    )(part, bias)


def kernel(x_, edge_index, W, att_src, att_dst, bias):
    x = x_.reshape(N, D)
    src = edge_index[0].astype(jnp.int32).reshape(NS, NCH, CH)
    dst = edge_index[1].astype(jnp.int32).reshape(NS, NCH, CH)
    att = jnp.concatenate(
        [att_src.reshape(1, D), att_dst.reshape(1, D)], axis=0)

    hpad, alpha2 = _embed(x, W, att)
    part = _edge(hpad, alpha2, src, dst)
    out = _combine(part, bias)
    return out.reshape(1, N, D)


# speedup vs baseline: 27.8983x; 1.1550x over previous
import jax
import jax.numpy as jnp
from jax import lax
from jax.experimental import pallas as pl
from jax.experimental.pallas import tpu as pltpu
from jax.experimental.pallas import tpu_sc as plsc

N = 10000
E = 320000
D = 128
DH = 64
NC = 2
NS = 16
EPW = E // NS
CH = 800
NCH = EPW // CH
RPT = N // NS
ZR = 125
ND = 10240
DT = ND // NS
L = 16


def _embed_body(x_ref, w_ref, att_ref, hpad_ref, alpha_ref):
    h = jnp.dot(x_ref[...], w_ref[...], preferred_element_type=jnp.float32)
    hpad_ref[0] = h[:, :DH]
    hpad_ref[1] = h[:, DH:]
    alpha_ref[...] = lax.dot_general(
        att_ref[...], h, (((1,), (1,)), ((), ())),
        preferred_element_type=jnp.float32)


def _embed(x, W, att):
    return pl.pallas_call(
        _embed_body,
        out_shape=[
            jax.ShapeDtypeStruct((NC, N, DH), jnp.float32),
            jax.ShapeDtypeStruct((2, N), jnp.float32),
        ],
    )(x, W, att)


def _edge_body(hpad_hbm, alpha_hbm, src_hbm, dst_hbm, out_hbm, den_hbm,
               asrc_v, adst_v, sidx_c, didx_c, w_c, rows_v, acc_sh, den_sh):
    c = lax.axis_index("c")
    s = lax.axis_index("s")

    def _zrow(r, _):
        for cc in range(DH // L):
            rows_v[r, pl.ds(cc * L, L)] = jnp.zeros((L,), jnp.float32)
        return ()
    lax.fori_loop(0, ZR, _zrow, ())
    for k in range(CH // L):
        w_c[pl.ds(k * L, L)] = jnp.zeros((L,), jnp.float32)
    zb = rows_v.at[pl.ds(0, ZR)]
    for t in range(RPT // ZR):
        pltpu.sync_copy(zb, acc_sh.at[pl.ds(s * RPT + t * ZR, ZR)])
    pltpu.sync_copy(w_c.at[pl.ds(0, DT)], den_sh.at[pl.ds(s * DT, DT)])

    pltpu.sync_copy(alpha_hbm.at[0], asrc_v)
    pltpu.sync_copy(alpha_hbm.at[1], adst_v)

    plsc.subcore_barrier()

    def _chunk(j, _):
        pltpu.sync_copy(src_hbm.at[s].at[j], sidx_c)
        pltpu.sync_copy(dst_hbm.at[s].at[j], didx_c)
        pltpu.sync_copy(hpad_hbm.at[c].at[sidx_c], rows_v)

        for k in range(CH // L):
            s16 = sidx_c[pl.ds(k * L, L)]
            d16 = didx_c[pl.ds(k * L, L)]
            al = (plsc.load_gather(asrc_v, [s16])
                  + plsc.load_gather(adst_v, [d16]))
            al = jnp.where(al >= 0.0, al, al * jnp.float32(0.2))
            w_c[pl.ds(k * L, L)] = jnp.exp(al)

        def _scale(r, _):
            wb = plsc.load_gather(w_c, [jnp.full((L,), r, jnp.int32)])
            for cc in range(DH // L):
                sl = pl.ds(cc * L, L)
                rows_v[r, sl] = rows_v[r, sl] * wb
            return ()
        lax.fori_loop(0, CH, _scale, ())

        pltpu.sync_copy(rows_v, acc_sh.at[didx_c], add=True)
        pltpu.sync_copy(w_c, den_sh.at[didx_c], add=True)
        return ()
    lax.fori_loop(0, NCH, _chunk, ())

    plsc.subcore_barrier()

    wb = rows_v.at[pl.ds(0, ZR)]
    for t in range(RPT // ZR):
        row0 = s * RPT + t * ZR
        pltpu.sync_copy(acc_sh.at[pl.ds(row0, ZR)], wb)
        pltpu.sync_copy(wb, out_hbm.at[c].at[pl.ds(row0, ZR)])
    pltpu.sync_copy(den_sh.at[pl.ds(s * DT, DT)], w_c.at[pl.ds(0, DT)])
    pltpu.sync_copy(w_c.at[pl.ds(0, DT)],
                    den_hbm.at[c].at[pl.ds(s * DT, DT)])


def _edge(hpad, alpha2, src_r, dst_r):
    mesh = plsc.VectorSubcoreMesh(
        core_axis_name="c", subcore_axis_name="s",
        num_cores=NC, num_subcores=NS)
    run = pl.kernel(
        _edge_body,
        out_type=[
            jax.ShapeDtypeStruct((NC, N, DH), jnp.float32),
            jax.ShapeDtypeStruct((NC, ND), jnp.float32),
        ],
        mesh=mesh,
        compiler_params=pltpu.CompilerParams(
            use_tc_tiling_on_sc=False, needs_layout_passes=False),
        scratch_types=[
            pltpu.VMEM((N,), jnp.float32),
            pltpu.VMEM((N,), jnp.float32),
            pltpu.VMEM((CH,), jnp.int32),
            pltpu.VMEM((CH,), jnp.int32),
            pltpu.VMEM((CH,), jnp.float32),
            pltpu.VMEM((CH, DH), jnp.float32),
            pltpu.VMEM_SHARED((N, DH), jnp.float32),
            pltpu.VMEM_SHARED((ND,), jnp.float32),
        ],
    )
    return run(hpad, alpha2, src_r, dst_r)


def _combine_body(part_ref, den_ref, bias_ref, out_ref):
    p = part_ref[...]
    d = den_ref[0, :N][:, None] + jnp.float32(1e-16)
    out_ref[...] = (jnp.concatenate([p[0] / d, p[1] / d], axis=1)
                    + bias_ref[...][None, :])


def _combine(part, den, bias):
    return pl.pallas_call(
        _combine_body,
        out_shape=jax.ShapeDtypeStruct((N, D), jnp.float32),
    )(part, den, bias)


def kernel(x_, edge_index, W, att_src, att_dst, bias):
    x = x_.reshape(N, D)
    src = edge_index[0].astype(jnp.int32).reshape(NS, NCH, CH)
    dst = edge_index[1].astype(jnp.int32).reshape(NS, NCH, CH)
    att = jnp.concatenate(
        [att_src.reshape(1, D), att_dst.reshape(1, D)], axis=0)

    hpad, alpha2 = _embed(x, W, att)
    part, den = _edge(hpad, alpha2, src, dst)
    out = _combine(part, den, bias)
    return out.reshape(1, N, D)


# speedup vs baseline: 30.6882x; 1.1000x over previous
import jax
import jax.numpy as jnp
from jax import lax
from jax.experimental import pallas as pl
from jax.experimental.pallas import tpu as pltpu
from jax.experimental.pallas import tpu_sc as plsc

N = 10000
E = 320000
D = 128
DH = 64
NC = 2
NS = 16
EPW = E // NS
CH = 400
NCH = EPW // CH
RPT = N // NS
ZR = 125
ND = 10240
DT = ND // NS
L = 16


def _embed_body(x_ref, w_ref, att_ref, hpad_ref, alpha_ref):
    h = jnp.dot(x_ref[...], w_ref[...], preferred_element_type=jnp.float32)
    hpad_ref[0] = h[:, :DH]
    hpad_ref[1] = h[:, DH:]
    alpha_ref[...] = lax.dot_general(
        att_ref[...], h, (((1,), (1,)), ((), ())),
        preferred_element_type=jnp.float32)


def _embed(x, W, att):
    return pl.pallas_call(
        _embed_body,
        out_shape=[
            jax.ShapeDtypeStruct((NC, N, DH), jnp.float32),
            jax.ShapeDtypeStruct((2, N), jnp.float32),
        ],
    )(x, W, att)


def _edge_body(hpad_hbm, alpha_hbm, src_hbm, dst_hbm, out_hbm, den_hbm,
               asrc_v, adst_v, sidx, didx, w2, rows2, sem0, sem1,
               acc_sh, den_sh):
    c = lax.axis_index("c")
    s = lax.axis_index("s")
    sems = (sem0, sem1)

    def _zrow(r, _):
        for cc in range(DH // L):
            rows2[0, r, pl.ds(cc * L, L)] = jnp.zeros((L,), jnp.float32)
        return ()
    lax.fori_loop(0, ZR, _zrow, ())
    for k in range(CH // L):
        w2[0, pl.ds(k * L, L)] = jnp.zeros((L,), jnp.float32)
    zb = rows2.at[0].at[pl.ds(0, ZR)]
    for t in range(RPT // ZR):
        pltpu.sync_copy(zb, acc_sh.at[pl.ds(s * RPT + t * ZR, ZR)])
    pltpu.sync_copy(w2.at[0], den_sh.at[pl.ds(s * DT, CH)])
    pltpu.sync_copy(w2.at[0].at[pl.ds(0, DT - CH)],
                    den_sh.at[pl.ds(s * DT + CH, DT - CH)])

    pltpu.sync_copy(alpha_hbm.at[0], asrc_v)
    pltpu.sync_copy(alpha_hbm.at[1], adst_v)

    plsc.subcore_barrier()

    pltpu.sync_copy(src_hbm.at[s].at[0], sidx.at[0])
    pltpu.sync_copy(dst_hbm.at[s].at[0], didx.at[0])
    pltpu.async_copy(hpad_hbm.at[c].at[sidx.at[0]], rows2.at[0], sem0)

    def _pair(i, _):
        for b in range(2):
            j = 2 * i + b
            nb = 1 - b

            def _prefetch():
                pltpu.sync_copy(src_hbm.at[s].at[j + 1], sidx.at[nb])
                pltpu.sync_copy(dst_hbm.at[s].at[j + 1], didx.at[nb])
                pltpu.async_copy(hpad_hbm.at[c].at[sidx.at[nb]],
                                 rows2.at[nb], sems[nb])
            if b == 0:
                _prefetch()
            else:
                pl.when(i < NCH // 2 - 1)(_prefetch)

            for k in range(CH // L):
                s16 = sidx[b, pl.ds(k * L, L)]
                d16 = didx[b, pl.ds(k * L, L)]
                al = (plsc.load_gather(asrc_v, [s16])
                      + plsc.load_gather(adst_v, [d16]))
                al = jnp.where(al >= 0.0, al, al * jnp.float32(0.2))
                w2[b, pl.ds(k * L, L)] = jnp.exp(al)

            pltpu.make_async_copy(hpad_hbm.at[c].at[sidx.at[b]],
                                  rows2.at[b], sems[b]).wait()

            def _scale(r, _):
                wb = plsc.load_gather(
                    w2.at[b], [jnp.full((L,), r, jnp.int32)])
                for cc in range(DH // L):
                    sl = pl.ds(cc * L, L)
                    rows2[b, r, sl] = rows2[b, r, sl] * wb
                return ()
            lax.fori_loop(0, CH, _scale, ())

            pltpu.sync_copy(rows2.at[b], acc_sh.at[didx.at[b]], add=True)
            pltpu.sync_copy(w2.at[b], den_sh.at[didx.at[b]], add=True)
        return ()
    lax.fori_loop(0, NCH // 2, _pair, ())

    plsc.subcore_barrier()

    wb = rows2.at[0].at[pl.ds(0, ZR)]
    for t in range(RPT // ZR):
        row0 = s * RPT + t * ZR
        pltpu.sync_copy(acc_sh.at[pl.ds(row0, ZR)], wb)
        pltpu.sync_copy(wb, out_hbm.at[c].at[pl.ds(row0, ZR)])
    pltpu.sync_copy(den_sh.at[pl.ds(s * DT, CH)], w2.at[0])
    pltpu.sync_copy(w2.at[0], den_hbm.at[c].at[pl.ds(s * DT, CH)])
    pltpu.sync_copy(den_sh.at[pl.ds(s * DT + CH, DT - CH)],
                    w2.at[0].at[pl.ds(0, DT - CH)])
    pltpu.sync_copy(w2.at[0].at[pl.ds(0, DT - CH)],
                    den_hbm.at[c].at[pl.ds(s * DT + CH, DT - CH)])


def _edge(hpad, alpha2, src_r, dst_r):
    mesh = plsc.VectorSubcoreMesh(
        core_axis_name="c", subcore_axis_name="s",
        num_cores=NC, num_subcores=NS)
    run = pl.kernel(
        _edge_body,
        out_type=[
            jax.ShapeDtypeStruct((NC, N, DH), jnp.float32),
            jax.ShapeDtypeStruct((NC, ND), jnp.float32),
        ],
        mesh=mesh,
        compiler_params=pltpu.CompilerParams(
            use_tc_tiling_on_sc=False, needs_layout_passes=False),
        scratch_types=[
            pltpu.VMEM((N,), jnp.float32),
            pltpu.VMEM((N,), jnp.float32),
            pltpu.VMEM((2, CH), jnp.int32),
            pltpu.VMEM((2, CH), jnp.int32),
            pltpu.VMEM((2, CH), jnp.float32),
            pltpu.VMEM((2, CH, DH), jnp.float32),
            pltpu.SemaphoreType.DMA,
            pltpu.SemaphoreType.DMA,
            pltpu.VMEM_SHARED((N, DH), jnp.float32),
            pltpu.VMEM_SHARED((ND,), jnp.float32),
        ],
    )
    return run(hpad, alpha2, src_r, dst_r)


def _combine_body(part_ref, den_ref, bias_ref, out_ref):
    p = part_ref[...]
    d = den_ref[0, :N][:, None] + jnp.float32(1e-16)
    out_ref[...] = (jnp.concatenate([p[0] / d, p[1] / d], axis=1)
                    + bias_ref[...][None, :])


def _combine(part, den, bias):
    return pl.pallas_call(
        _combine_body,
        out_shape=jax.ShapeDtypeStruct((N, D), jnp.float32),
    )(part, den, bias)


def kernel(x_, edge_index, W, att_src, att_dst, bias):
    x = x_.reshape(N, D)
    src = edge_index[0].astype(jnp.int32).reshape(NS, NCH, CH)
    dst = edge_index[1].astype(jnp.int32).reshape(NS, NCH, CH)
    att = jnp.concatenate(
        [att_src.reshape(1, D), att_dst.reshape(1, D)], axis=0)

    hpad, alpha2 = _embed(x, W, att)
    part, den = _edge(hpad, alpha2, src, dst)
    out = _combine(part, den, bias)
    return out.reshape(1, N, D)


# speedup vs baseline: 32.4392x; 1.0571x over previous
import jax
import jax.numpy as jnp
from jax import lax
from jax.experimental import pallas as pl
from jax.experimental.pallas import tpu as pltpu
from jax.experimental.pallas import tpu_sc as plsc

N = 10000
E = 320000
D = 128
DH = 64
NC = 2
NS = 16
EPW = E // NS
CH = 400
NCH = EPW // CH
RPT = N // NS
ZR = 125
ND = 10240
DT = ND // NS
L = 16


def _embed_body(x_ref, w_ref, att_ref, hpad_ref, alpha_ref):
    h = jnp.dot(x_ref[...], w_ref[...], preferred_element_type=jnp.float32)
    hpad_ref[0] = h[:, :DH]
    hpad_ref[1] = h[:, DH:]
    alpha_ref[...] = lax.dot_general(
        att_ref[...], h, (((1,), (1,)), ((), ())),
        preferred_element_type=jnp.float32)


def _embed(x, W, att):
    return pl.pallas_call(
        _embed_body,
        out_shape=[
            jax.ShapeDtypeStruct((NC, N, DH), jnp.float32),
            jax.ShapeDtypeStruct((2, N), jnp.float32),
        ],
    )(x, W, att)


def _edge_body(hpad_hbm, alpha_hbm, src_hbm, dst_hbm, out_hbm, den_hbm,
               asrc_v, adst_v, sidx, didx, w2, rows2, sem0, sem1,
               acc_sh, den_sh):
    c = lax.axis_index("c")
    s = lax.axis_index("s")
    sems = (sem0, sem1)

    def _zrow(r, _):
        for cc in range(DH // L):
            rows2[0, r, pl.ds(cc * L, L)] = jnp.zeros((L,), jnp.float32)
        return ()
    lax.fori_loop(0, ZR, _zrow, ())
    for k in range(CH // L):
        w2[0, pl.ds(k * L, L)] = jnp.zeros((L,), jnp.float32)
    zb = rows2.at[0].at[pl.ds(0, ZR)]
    for t in range(RPT // ZR):
        pltpu.sync_copy(zb, acc_sh.at[pl.ds(s * RPT + t * ZR, ZR)])
    pltpu.sync_copy(w2.at[0], den_sh.at[pl.ds(s * DT, CH)])
    pltpu.sync_copy(w2.at[0].at[pl.ds(0, DT - CH)],
                    den_sh.at[pl.ds(s * DT + CH, DT - CH)])

    pltpu.sync_copy(alpha_hbm.at[0], asrc_v)
    pltpu.sync_copy(alpha_hbm.at[1], adst_v)

    plsc.subcore_barrier()

    pltpu.sync_copy(src_hbm.at[s].at[0], sidx.at[0])
    pltpu.sync_copy(dst_hbm.at[s].at[0], didx.at[0])
    pltpu.async_copy(hpad_hbm.at[c].at[sidx.at[0]], rows2.at[0], sem0)

    def _pair(i, _):
        for b in range(2):
            j = 2 * i + b
            nb = 1 - b

            def _prefetch():
                pltpu.sync_copy(src_hbm.at[s].at[j + 1], sidx.at[nb])
                pltpu.sync_copy(dst_hbm.at[s].at[j + 1], didx.at[nb])
                pltpu.async_copy(hpad_hbm.at[c].at[sidx.at[nb]],
                                 rows2.at[nb], sems[nb])
            if b == 0:
                _prefetch()
            else:
                pl.when(i < NCH // 2 - 1)(_prefetch)

            for k in range(CH // L):
                s16 = sidx[b, pl.ds(k * L, L)]
                d16 = didx[b, pl.ds(k * L, L)]
                al = (plsc.load_gather(asrc_v, [s16])
                      + plsc.load_gather(adst_v, [d16]))
                al = jnp.where(al >= 0.0, al, al * jnp.float32(0.2))
                w2[b, pl.ds(k * L, L)] = jnp.exp(al)

            pltpu.make_async_copy(hpad_hbm.at[c].at[sidx.at[b]],
                                  rows2.at[b], sems[b]).wait()

            def _scale(r4, _):
                r0 = r4 * 4
                for rr in range(4):
                    r = r0 + rr
                    wb = plsc.load_gather(
                        w2.at[b], [jnp.full((L,), r, jnp.int32)])
                    for cc in range(DH // L):
                        sl = pl.ds(cc * L, L)
                        rows2[b, r, sl] = rows2[b, r, sl] * wb
                return ()
            lax.fori_loop(0, CH // 4, _scale, ())

            pltpu.sync_copy(rows2.at[b], acc_sh.at[didx.at[b]], add=True)
            pltpu.sync_copy(w2.at[b], den_sh.at[didx.at[b]], add=True)
        return ()
    lax.fori_loop(0, NCH // 2, _pair, ())

    plsc.subcore_barrier()

    wb = rows2.at[0].at[pl.ds(0, ZR)]
    for t in range(RPT // ZR):
        row0 = s * RPT + t * ZR
        pltpu.sync_copy(acc_sh.at[pl.ds(row0, ZR)], wb)
        pltpu.sync_copy(wb, out_hbm.at[c].at[pl.ds(row0, ZR)])
    pltpu.sync_copy(den_sh.at[pl.ds(s * DT, CH)], w2.at[0])
    pltpu.sync_copy(w2.at[0], den_hbm.at[c].at[pl.ds(s * DT, CH)])
    pltpu.sync_copy(den_sh.at[pl.ds(s * DT + CH, DT - CH)],
                    w2.at[0].at[pl.ds(0, DT - CH)])
    pltpu.sync_copy(w2.at[0].at[pl.ds(0, DT - CH)],
                    den_hbm.at[c].at[pl.ds(s * DT + CH, DT - CH)])


def _edge(hpad, alpha2, src_r, dst_r):
    mesh = plsc.VectorSubcoreMesh(
        core_axis_name="c", subcore_axis_name="s",
        num_cores=NC, num_subcores=NS)
    run = pl.kernel(
        _edge_body,
        out_type=[
            jax.ShapeDtypeStruct((NC, N, DH), jnp.float32),
            jax.ShapeDtypeStruct((NC, ND), jnp.float32),
        ],
        mesh=mesh,
        compiler_params=pltpu.CompilerParams(
            use_tc_tiling_on_sc=False, needs_layout_passes=False),
        scratch_types=[
            pltpu.VMEM((N,), jnp.float32),
            pltpu.VMEM((N,), jnp.float32),
            pltpu.VMEM((2, CH), jnp.int32),
            pltpu.VMEM((2, CH), jnp.int32),
            pltpu.VMEM((2, CH), jnp.float32),
            pltpu.VMEM((2, CH, DH), jnp.float32),
            pltpu.SemaphoreType.DMA,
            pltpu.SemaphoreType.DMA,
            pltpu.VMEM_SHARED((N, DH), jnp.float32),
            pltpu.VMEM_SHARED((ND,), jnp.float32),
        ],
    )
    return run(hpad, alpha2, src_r, dst_r)


def _combine_body(part_ref, den_ref, bias_ref, out_ref):
    p = part_ref[...]
    d = den_ref[0, :N][:, None] + jnp.float32(1e-16)
    out_ref[...] = (jnp.concatenate([p[0] / d, p[1] / d], axis=1)
                    + bias_ref[...][None, :])


def _combine(part, den, bias):
    return pl.pallas_call(
        _combine_body,
        out_shape=jax.ShapeDtypeStruct((N, D), jnp.float32),
    )(part, den, bias)


def kernel(x_, edge_index, W, att_src, att_dst, bias):
    x = x_.reshape(N, D)
    src = edge_index[0].astype(jnp.int32).reshape(NS, NCH, CH)
    dst = edge_index[1].astype(jnp.int32).reshape(NS, NCH, CH)
    att = jnp.concatenate(
        [att_src.reshape(1, D), att_dst.reshape(1, D)], axis=0)

    hpad, alpha2 = _embed(x, W, att)
    part, den = _edge(hpad, alpha2, src, dst)
    out = _combine(part, den, bias)
    return out.reshape(1, N, D)
